# baseline (speedup 1.0000x reference)
import functools

import jax
import jax.numpy as jnp
from jax import lax
from jax.experimental import pallas as pl
from jax.experimental.pallas import tpu as pltpu
from jax.experimental.pallas import tpu_sc as plsc

N_NODES = 10000
N_PAD = 10240
D_IN = 128
H1 = 8
F1 = 8
C_OUT = 7
E_EDGES = 320000
E_REAL = E_EDGES + N_NODES
NW = 32
CHUNK = 128
N_CHUNKS = 82
EW = N_CHUNKS * CHUNK
E_PAD = EW * NW
TILES = 16
ROWS_PER_TILE = N_PAD // TILES
BLK = 1280


def _stage_a(x_pad, w1s, w1d):
    def body(x_ref, ws_ref, wd_ref, s_ref, d_ref):
        xb = x_ref[...]
        s_ref[...] = jnp.dot(xb, ws_ref[...], preferred_element_type=jnp.float32)
        d_ref[...] = jnp.dot(xb, wd_ref[...], preferred_element_type=jnp.float32)

    return pl.pallas_call(
        body,
        grid=(N_PAD // BLK,),
        in_specs=[
            pl.BlockSpec((BLK, D_IN), lambda i: (i, 0)),
            pl.BlockSpec((D_IN, 80), lambda i: (0, 0)),
            pl.BlockSpec((D_IN, 16), lambda i: (0, 0)),
        ],
        out_specs=[
            pl.BlockSpec((BLK, 80), lambda i: (i, 0)),
            pl.BlockSpec((BLK, 16), lambda i: (i, 0)),
        ],
        out_shape=[
            jax.ShapeDtypeStruct((N_PAD, 80), jnp.float32),
            jax.ShapeDtypeStruct((N_PAD, 16), jnp.float32),
        ],
    )(x_pad, w1s, w1d)


def _sc_layer1(src, dst, s_tab, d_tab):
    mesh = plsc.VectorSubcoreMesh(core_axis_name="c", subcore_axis_name="s")

    @functools.partial(
        pl.kernel,
        out_type=[
            jax.ShapeDtypeStruct((2, N_PAD, 64), jnp.float32),
            jax.ShapeDtypeStruct((2, N_PAD, 16), jnp.float32),
        ],
        mesh=mesh,
        scratch_types=[
            pltpu.VMEM((N_CHUNKS, CHUNK), jnp.int32),
            pltpu.VMEM((N_CHUNKS, CHUNK), jnp.int32),
            [pltpu.VMEM((CHUNK, 80), jnp.float32) for _ in range(2)],
            [pltpu.VMEM((CHUNK, 16), jnp.float32) for _ in range(2)],
            [pltpu.VMEM((CHUNK, 16), jnp.float32) for _ in range(2)],
            [pltpu.VMEM((CHUNK, 64), jnp.float32) for _ in range(2)],
            pltpu.VMEM_SHARED((N_PAD, 64), jnp.float32),
            pltpu.VMEM_SHARED((N_PAD, 16), jnp.float32),
            [pltpu.SemaphoreType.DMA for _ in range(2)],
            [pltpu.SemaphoreType.DMA for _ in range(2)],
        ],
        compiler_params=pltpu.CompilerParams(needs_layout_passes=False, use_tc_tiling_on_sc=False),
    )
    def k(src_h, dst_h, s_h, d_h, acc_h, den_h,
          sidx, didx, srows, drows, wbuf, msg, acc_sp, den_sp, semg, sems):
        cid = lax.axis_index("c")
        sid = lax.axis_index("s")
        wid = cid * TILES + sid
        iota = lax.iota(jnp.int32, 16)
        lo8 = lax.bitwise_and(iota, 7)
        hi8 = lax.shift_right_logical(iota, 3)
        zero16 = jnp.zeros((16,), jnp.float32)

        def zb(i, c):
            for p in range(2):
                wbuf[p][i, pl.ds(0, 16)] = zero16
                for kk in range(4):
                    msg[p][i, pl.ds(16 * kk, 16)] = zero16
            return c

        lax.fori_loop(0, CHUNK, zb, 0)
        base_r = sid * ROWS_PER_TILE
        for j in range(ROWS_PER_TILE // CHUNK):
            pltpu.sync_copy(msg[0], acc_sp.at[pl.ds(base_r + j * CHUNK, CHUNK)])
            pltpu.sync_copy(wbuf[0], den_sp.at[pl.ds(base_r + j * CHUNK, CHUNK)])
        pltpu.sync_copy(src_h.at[wid], sidx)
        pltpu.sync_copy(dst_h.at[wid], didx)
        plsc.subcore_barrier()

        def fire_gather(ci, p):
            pltpu.async_copy(s_h.at[sidx.at[ci]], srows[p], semg[p])
            pltpu.async_copy(d_h.at[didx.at[ci]], drows[p], semg[p])

        def wait_gather(ci, p):
            pltpu.make_async_copy(s_h.at[sidx.at[ci]], srows[p], semg[p]).wait()
            pltpu.make_async_copy(d_h.at[didx.at[ci]], drows[p], semg[p]).wait()

        def fire_scatter(ci, p):
            pltpu.async_copy(msg[p], acc_sp.at[didx.at[ci]], sems[p], add=True)
            pltpu.async_copy(wbuf[p], den_sp.at[didx.at[ci]], sems[p], add=True)

        def wait_scatter(ci, p):
            pltpu.make_async_copy(msg[p], acc_sp.at[didx.at[ci]], sems[p]).wait()
            pltpu.make_async_copy(wbuf[p], den_sp.at[didx.at[ci]], sems[p]).wait()

        def compute(p):
            @plsc.parallel_loop(0, 64, unroll=4)
            def wbody(t):
                rvec = 2 * t + hi8
                a_s = plsc.load_gather(srows[p], [rvec, 64 + lo8])
                a_d = plsc.load_gather(drows[p], [rvec, lo8])
                e = a_s + a_d
                e = jnp.where(e >= 0.0, e, 0.2 * e)
                plsc.store_scatter(wbuf[p], [rvec, lo8], jnp.exp(e))

            @plsc.parallel_loop(0, CHUNK, unroll=8)
            def mbody(b):
                bfull = jnp.full((16,), 0, jnp.int32) + b
                for kk in range(4):
                    hvec = srows[p][b, pl.ds(16 * kk, 16)]
                    w16 = plsc.load_gather(wbuf[p], [bfull, 2 * kk + hi8])
                    msg[p][b, pl.ds(16 * kk, 16)] = hvec * w16

        for p in range(2):
            fire_gather(p, p)
        for p in range(2):
            wait_gather(p, p)
            compute(p)
            fire_scatter(p, p)
            fire_gather(p + 2, p)

        def outer(g, c):
            for p in range(2):
                ci = 2 * g + p
                wait_gather(ci, p)
                wait_scatter(ci - 2, p)
                compute(p)
                fire_scatter(ci, p)
                fire_gather(ci + 2, p)
            return c

        lax.fori_loop(1, N_CHUNKS // 2 - 1, outer, 0)
        for p in range(2):
            ci = N_CHUNKS - 2 + p
            wait_gather(ci, p)
            wait_scatter(ci - 2, p)
            compute(p)
            fire_scatter(ci, p)
        for p in range(2):
            wait_scatter(N_CHUNKS - 2 + p, p)
        plsc.subcore_barrier()
        pltpu.sync_copy(acc_sp.at[pl.ds(base_r, ROWS_PER_TILE)],
                        acc_h.at[cid, pl.ds(base_r, ROWS_PER_TILE)])
        pltpu.sync_copy(den_sp.at[pl.ds(base_r, ROWS_PER_TILE)],
                        den_h.at[cid, pl.ds(base_r, ROWS_PER_TILE)])

    return k(src, dst, s_tab, d_tab)


def _stage_c(acc, den, bias1, r16, w2c, e7):
    def body(acc_ref, den_ref, b1_ref, r16_ref, w2c_ref, e7_ref, t2_ref):
        a = acc_ref[0] + acc_ref[1]
        dn = den_ref[0] + den_ref[1]
        dexp = jnp.dot(dn, r16_ref[...], preferred_element_type=jnp.float32) + 1e-16
        h1 = a / dexp + b1_ref[...]
        h1 = jnp.where(h1 > 0.0, h1, jnp.exp(jnp.minimum(h1, 0.0)) - 1.0)
        t2_ref[...] = jnp.dot(h1, w2c_ref[...],
                              preferred_element_type=jnp.float32) + e7_ref[...]

    return pl.pallas_call(
        body,
        grid=(N_PAD // BLK,),
        in_specs=[
            pl.BlockSpec((2, BLK, 64), lambda i: (0, i, 0)),
            pl.BlockSpec((2, BLK, 16), lambda i: (0, i, 0)),
            pl.BlockSpec((1, 64), lambda i: (0, 0)),
            pl.BlockSpec((16, 64), lambda i: (0, 0)),
            pl.BlockSpec((64, 16), lambda i: (0, 0)),
            pl.BlockSpec((1, 16), lambda i: (0, 0)),
        ],
        out_specs=pl.BlockSpec((BLK, 16), lambda i: (i, 0)),
        out_shape=jax.ShapeDtypeStruct((N_PAD, 16), jnp.float32),
    )(acc, den, bias1, r16, w2c, e7)


def _sc_layer2(src, dst, t2_tab):
    mesh = plsc.VectorSubcoreMesh(core_axis_name="c", subcore_axis_name="s")

    @functools.partial(
        pl.kernel,
        out_type=jax.ShapeDtypeStruct((2, N_PAD, 16), jnp.float32),
        mesh=mesh,
        scratch_types=[
            pltpu.VMEM((N_CHUNKS, CHUNK), jnp.int32),
            pltpu.VMEM((N_CHUNKS, CHUNK), jnp.int32),
            [pltpu.VMEM((CHUNK, 16), jnp.float32) for _ in range(2)],
            [pltpu.VMEM((CHUNK, 16), jnp.float32) for _ in range(2)],
            [pltpu.VMEM((CHUNK,), jnp.float32) for _ in range(2)],
            [pltpu.VMEM((CHUNK, 16), jnp.float32) for _ in range(2)],
            pltpu.VMEM_SHARED((N_PAD, 16), jnp.float32),
            [pltpu.SemaphoreType.DMA for _ in range(2)],
            [pltpu.SemaphoreType.DMA for _ in range(2)],
        ],
        compiler_params=pltpu.CompilerParams(needs_layout_passes=False, use_tc_tiling_on_sc=False),
    )
    def k(src_h, dst_h, t2_h, acc_h,
          sidx, didx, s2, d2, wv, msg, acc_sp, semg, sems):
        cid = lax.axis_index("c")
        sid = lax.axis_index("s")
        wid = cid * TILES + sid
        iota = lax.iota(jnp.int32, 16)
        col8 = jnp.full((16,), 0, jnp.int32) + 8
        col9 = col8 + 1
        zero16 = jnp.zeros((16,), jnp.float32)

        def zb(i, c):
            msg[0][i, pl.ds(0, 16)] = zero16
            msg[1][i, pl.ds(0, 16)] = zero16
            return c

        lax.fori_loop(0, CHUNK, zb, 0)
        base_r = sid * ROWS_PER_TILE
        for j in range(ROWS_PER_TILE // CHUNK):
            pltpu.sync_copy(msg[0], acc_sp.at[pl.ds(base_r + j * CHUNK, CHUNK)])
        pltpu.sync_copy(src_h.at[wid], sidx)
        pltpu.sync_copy(dst_h.at[wid], didx)
        plsc.subcore_barrier()

        def fire_gather(ci, p):
            pltpu.async_copy(t2_h.at[sidx.at[ci]], s2[p], semg[p])
            pltpu.async_copy(t2_h.at[didx.at[ci]], d2[p], semg[p])

        def wait_gather(ci, p):
            pltpu.make_async_copy(t2_h.at[sidx.at[ci]], s2[p], semg[p]).wait()
            pltpu.make_async_copy(t2_h.at[didx.at[ci]], d2[p], semg[p]).wait()

        def fire_scatter(ci, p):
            pltpu.async_copy(msg[p], acc_sp.at[didx.at[ci]], sems[p], add=True)

        def wait_scatter(ci, p):
            pltpu.make_async_copy(msg[p], acc_sp.at[didx.at[ci]], sems[p]).wait()

        def compute(p):
            @plsc.parallel_loop(0, CHUNK // 16, unroll=4)
            def wbody(t):
                rvec = t * 16 + iota
                a_s = plsc.load_gather(s2[p], [rvec, col8])
                a_d = plsc.load_gather(d2[p], [rvec, col9])
                e = a_s + a_d
                e = jnp.where(e >= 0.0, e, 0.2 * e)
                wv[p][pl.ds(t * 16, 16)] = jnp.exp(e)

            @plsc.parallel_loop(0, CHUNK, unroll=8)
            def mbody(b):
                bfull = jnp.full((16,), 0, jnp.int32) + b
                srow = s2[p][b, pl.ds(0, 16)]
                w16 = plsc.load_gather(wv[p], [bfull])
                msg[p][b, pl.ds(0, 16)] = srow * w16

        for p in range(2):
            fire_gather(p, p)
        for p in range(2):
            wait_gather(p, p)
            compute(p)
            fire_scatter(p, p)
            fire_gather(p + 2, p)

        def outer(g, c):
            for p in range(2):
                ci = 2 * g + p
                wait_gather(ci, p)
                wait_scatter(ci - 2, p)
                compute(p)
                fire_scatter(ci, p)
                fire_gather(ci + 2, p)
            return c

        lax.fori_loop(1, N_CHUNKS // 2 - 1, outer, 0)
        for p in range(2):
            ci = N_CHUNKS - 2 + p
            wait_gather(ci, p)
            wait_scatter(ci - 2, p)
            compute(p)
            fire_scatter(ci, p)
        for p in range(2):
            wait_scatter(N_CHUNKS - 2 + p, p)
        plsc.subcore_barrier()
        pltpu.sync_copy(acc_sp.at[pl.ds(base_r, ROWS_PER_TILE)],
                        acc_h.at[cid, pl.ds(base_r, ROWS_PER_TILE)])

    return k(src, dst, t2_tab)


def _stage_e(acc2, bias2p):
    def body(acc_ref, b2_ref, o_ref):
        o = acc_ref[0] + acc_ref[1]
        d = o[:, 7:8]
        v = o / (d + 1e-16) + b2_ref[...]
        lane = lax.broadcasted_iota(jnp.int32, (BLK, 16), 1)
        v = jnp.where(lane < C_OUT, v, -1e30)
        m = jnp.max(v, axis=1, keepdims=True)
        s = jnp.log(jnp.sum(jnp.exp(v - m), axis=1, keepdims=True))
        o_ref[...] = v - m - s

    return pl.pallas_call(
        body,
        grid=(N_PAD // BLK,),
        in_specs=[
            pl.BlockSpec((2, BLK, 16), lambda i: (0, i, 0)),
            pl.BlockSpec((1, 16), lambda i: (0, 0)),
        ],
        out_specs=pl.BlockSpec((BLK, 16), lambda i: (i, 0)),
        out_shape=jax.ShapeDtypeStruct((N_PAD, 16), jnp.float32),
    )(acc2, bias2p)


def kernel(x, edge_index, W1, att_src1, att_dst1, bias1,
           W2, att_src2, att_dst2, bias2):
    loop = jnp.arange(N_NODES, dtype=jnp.int32)
    npad = E_PAD - E_REAL
    pad_idx = N_NODES + (jnp.arange(npad, dtype=jnp.int32) % (N_PAD - N_NODES))
    src = jnp.concatenate([edge_index[0].astype(jnp.int32), loop, pad_idx]
                          ).reshape(N_CHUNKS, NW, CHUNK).swapaxes(0, 1)
    dst = jnp.concatenate([edge_index[1].astype(jnp.int32), loop, pad_idx]
                          ).reshape(N_CHUNKS, NW, CHUNK).swapaxes(0, 1)
    x_pad = jnp.zeros((N_PAD, D_IN), jnp.float32).at[:N_NODES].set(x)

    w1r = W1.reshape(D_IN, H1, F1)
    a_src_w = jnp.einsum("dhf,hf->dh", w1r, att_src1)
    a_dst_w = jnp.einsum("dhf,hf->dh", w1r, att_dst1)
    zpad8 = jnp.zeros((D_IN, 8), jnp.float32)
    w1s = jnp.concatenate([W1, a_src_w, zpad8], axis=1)
    w1d = jnp.concatenate([a_dst_w, zpad8], axis=1)

    s_tab, d_tab = _stage_a(x_pad, w1s, w1d)
    acc, den = _sc_layer1(src, dst, s_tab, d_tab)

    r16 = jnp.zeros((16, 64), jnp.float32).at[:8].set(
        jnp.repeat(jnp.eye(8, dtype=jnp.float32), 8, axis=1))
    w2c = jnp.zeros((64, 16), jnp.float32)
    w2c = w2c.at[:, :C_OUT].set(W2)
    w2c = w2c.at[:, 8].set(W2 @ att_src2[0])
    w2c = w2c.at[:, 9].set(W2 @ att_dst2[0])
    e7 = jnp.zeros((1, 16), jnp.float32).at[0, 7].set(1.0)

    t2_tab = _stage_c(acc, den, bias1.reshape(1, 64), r16, w2c, e7)
    acc2 = _sc_layer2(src, dst, t2_tab)
    out = _stage_e(acc2, jnp.pad(bias2, (0, 16 - C_OUT)).reshape(1, 16))
    return out[:N_NODES, :C_OUT]


# speedup vs baseline: 1.0166x; 1.0166x over previous
import functools

import jax
import jax.numpy as jnp
from jax import lax
from jax.experimental import pallas as pl
from jax.experimental.pallas import tpu as pltpu
from jax.experimental.pallas import tpu_sc as plsc

N_NODES = 10000
N_PAD = 10240
D_IN = 128
H1 = 8
F1 = 8
C_OUT = 7
E_EDGES = 320000
E_REAL = E_EDGES + N_NODES
NW = 32
CHUNK = 128
N_CHUNKS = 82
EW = N_CHUNKS * CHUNK
E_PAD = EW * NW
TILES = 16
ROWS_PER_TILE = N_PAD // TILES
BLK = 1280


def _stage_a(x_pad, w1s, w1d):
    def body(x_ref, ws_ref, wd_ref, s_ref, d_ref):
        xb = x_ref[...]
        s_ref[...] = jnp.dot(xb, ws_ref[...], preferred_element_type=jnp.float32)
        d_ref[...] = jnp.dot(xb, wd_ref[...], preferred_element_type=jnp.float32)

    return pl.pallas_call(
        body,
        grid=(N_PAD // BLK,),
        in_specs=[
            pl.BlockSpec((BLK, D_IN), lambda i: (i, 0)),
            pl.BlockSpec((D_IN, 80), lambda i: (0, 0)),
            pl.BlockSpec((D_IN, 16), lambda i: (0, 0)),
        ],
        out_specs=[
            pl.BlockSpec((BLK, 80), lambda i: (i, 0)),
            pl.BlockSpec((BLK, 16), lambda i: (i, 0)),
        ],
        out_shape=[
            jax.ShapeDtypeStruct((N_PAD, 80), jnp.float32),
            jax.ShapeDtypeStruct((N_PAD, 16), jnp.float32),
        ],
    )(x_pad, w1s, w1d)


def _sc_layer1(src, dst, s_tab, d_tab):
    mesh = plsc.VectorSubcoreMesh(core_axis_name="c", subcore_axis_name="s")

    @functools.partial(
        pl.kernel,
        out_type=jax.ShapeDtypeStruct((2, N_PAD, 80), jnp.float32),
        mesh=mesh,
        scratch_types=[
            pltpu.VMEM((N_CHUNKS, CHUNK), jnp.int32),
            pltpu.VMEM((N_CHUNKS, CHUNK), jnp.int32),
            [pltpu.VMEM((CHUNK, 80), jnp.float32) for _ in range(2)],
            [pltpu.VMEM((CHUNK, 16), jnp.float32) for _ in range(2)],
            [pltpu.VMEM((CHUNK, 80), jnp.float32) for _ in range(2)],
            pltpu.VMEM_SHARED((N_PAD, 80), jnp.float32),
            [pltpu.SemaphoreType.DMA for _ in range(2)],
            [pltpu.SemaphoreType.DMA for _ in range(2)],
            pltpu.SemaphoreType.DMA,
        ],
        compiler_params=pltpu.CompilerParams(needs_layout_passes=False, use_tc_tiling_on_sc=False),
    )
    def k(src_h, dst_h, s_h, d_h, acc_h,
          sidx, didx, srows, drows, msg, acc_sp, semg, sems, semp):
        cid = lax.axis_index("c")
        sid = lax.axis_index("s")
        wid = cid * TILES + sid
        iota = lax.iota(jnp.int32, 16)
        lo8 = lax.bitwise_and(iota, 7)
        hi8 = lax.shift_right_logical(iota, 3)
        zero16 = jnp.zeros((16,), jnp.float32)

        def zb(i, c):
            for p in range(2):
                for kk in range(5):
                    msg[p][i, pl.ds(16 * kk, 16)] = zero16
            return c

        lax.fori_loop(0, CHUNK, zb, 0)
        base_r = sid * ROWS_PER_TILE
        for j in range(ROWS_PER_TILE // CHUNK):
            pltpu.sync_copy(msg[0], acc_sp.at[pl.ds(base_r + j * CHUNK, CHUNK)])
        pltpu.sync_copy(src_h.at[wid], sidx)
        pltpu.sync_copy(dst_h.at[wid], didx)
        plsc.subcore_barrier()

        def fire_gather(ci, p):
            pltpu.async_copy(s_h.at[sidx.at[ci]], srows[p], semg[p])
            pltpu.async_copy(d_h.at[didx.at[ci]], drows[p], semg[p])

        def wait_gather(ci, p):
            pltpu.make_async_copy(s_h.at[sidx.at[ci]], srows[p], semg[p]).wait()
            pltpu.make_async_copy(d_h.at[didx.at[ci]], drows[p], semg[p]).wait()

        def fire_scatter(ci, p):
            pltpu.async_copy(msg[p], acc_sp.at[didx.at[ci]], sems[p], add=True)

        def wait_scatter(ci, p):
            pltpu.make_async_copy(msg[p], acc_sp.at[didx.at[ci]], sems[p]).wait()

        def compute(p):
            @plsc.parallel_loop(0, 64, unroll=4)
            def wbody(t):
                rvec = 2 * t + hi8
                a_s = plsc.load_gather(srows[p], [rvec, 64 + lo8])
                a_d = plsc.load_gather(drows[p], [rvec, lo8])
                e = a_s + a_d
                e = jnp.where(e >= 0.0, e, 0.2 * e)
                plsc.store_scatter(msg[p], [rvec, 64 + lo8], jnp.exp(e))

            @plsc.parallel_loop(0, CHUNK, unroll=8)
            def mbody(b):
                bfull = jnp.full((16,), 0, jnp.int32) + b
                for kk in range(4):
                    hvec = srows[p][b, pl.ds(16 * kk, 16)]
                    w16 = plsc.load_gather(msg[p], [bfull, 64 + 2 * kk + hi8])
                    msg[p][b, pl.ds(16 * kk, 16)] = hvec * w16

        for p in range(2):
            fire_gather(p, p)
        for p in range(2):
            wait_gather(p, p)
            compute(p)
            fire_scatter(p, p)
            fire_gather(p + 2, p)

        def outer(g, c):
            for p in range(2):
                ci = 2 * g + p
                wait_gather(ci, p)
                wait_scatter(ci - 2, p)
                compute(p)
                fire_scatter(ci, p)
                fire_gather(ci + 2, p)
            return c

        lax.fori_loop(1, N_CHUNKS // 2 - 1, outer, 0)
        for p in range(2):
            ci = N_CHUNKS - 2 + p
            wait_gather(ci, p)
            wait_scatter(ci - 2, p)
            compute(p)
            fire_scatter(ci, p)
        for p in range(2):
            wait_scatter(N_CHUNKS - 2 + p, p)
        plsc.subcore_barrier()
        pltpu.sync_copy(acc_sp.at[pl.ds(base_r, ROWS_PER_TILE)],
                        acc_h.at[cid, pl.ds(base_r, ROWS_PER_TILE)])

    return k(src, dst, s_tab, d_tab)


def _stage_c(acc, bias1, r16, w2c, e7):
    def body(acc_ref, b1_ref, r16_ref, w2c_ref, e7_ref, t2_ref):
        a = acc_ref[0, :, :64] + acc_ref[1, :, :64]
        dn = acc_ref[0, :, 64:80] + acc_ref[1, :, 64:80]
        dexp = jnp.dot(dn, r16_ref[...], preferred_element_type=jnp.float32) + 1e-16
        h1 = a / dexp + b1_ref[...]
        h1 = jnp.where(h1 > 0.0, h1, jnp.exp(jnp.minimum(h1, 0.0)) - 1.0)
        t2_ref[...] = jnp.dot(h1, w2c_ref[...],
                              preferred_element_type=jnp.float32) + e7_ref[...]

    return pl.pallas_call(
        body,
        grid=(N_PAD // BLK,),
        in_specs=[
            pl.BlockSpec((2, BLK, 80), lambda i: (0, i, 0)),
            pl.BlockSpec((1, 64), lambda i: (0, 0)),
            pl.BlockSpec((16, 64), lambda i: (0, 0)),
            pl.BlockSpec((64, 16), lambda i: (0, 0)),
            pl.BlockSpec((1, 16), lambda i: (0, 0)),
        ],
        out_specs=pl.BlockSpec((BLK, 16), lambda i: (i, 0)),
        out_shape=jax.ShapeDtypeStruct((N_PAD, 16), jnp.float32),
    )(acc, bias1, r16, w2c, e7)


def _sc_layer2(src, dst, t2_tab):
    mesh = plsc.VectorSubcoreMesh(core_axis_name="c", subcore_axis_name="s")

    @functools.partial(
        pl.kernel,
        out_type=jax.ShapeDtypeStruct((2, N_PAD, 16), jnp.float32),
        mesh=mesh,
        scratch_types=[
            pltpu.VMEM((N_CHUNKS, CHUNK), jnp.int32),
            pltpu.VMEM((N_CHUNKS, CHUNK), jnp.int32),
            [pltpu.VMEM((CHUNK, 16), jnp.float32) for _ in range(2)],
            [pltpu.VMEM((CHUNK, 16), jnp.float32) for _ in range(2)],
            [pltpu.VMEM((CHUNK,), jnp.float32) for _ in range(2)],
            [pltpu.VMEM((CHUNK, 16), jnp.float32) for _ in range(2)],
            pltpu.VMEM_SHARED((N_PAD, 16), jnp.float32),
            [pltpu.SemaphoreType.DMA for _ in range(2)],
            [pltpu.SemaphoreType.DMA for _ in range(2)],
            pltpu.SemaphoreType.DMA,
        ],
        compiler_params=pltpu.CompilerParams(needs_layout_passes=False, use_tc_tiling_on_sc=False),
    )
    def k(src_h, dst_h, t2_h, acc_h,
          sidx, didx, s2, d2, wv, msg, acc_sp, semg, sems, semp):
        cid = lax.axis_index("c")
        sid = lax.axis_index("s")
        wid = cid * TILES + sid
        iota = lax.iota(jnp.int32, 16)
        col8 = jnp.full((16,), 0, jnp.int32) + 8
        col9 = col8 + 1
        zero16 = jnp.zeros((16,), jnp.float32)

        def zb(i, c):
            msg[0][i, pl.ds(0, 16)] = zero16
            msg[1][i, pl.ds(0, 16)] = zero16
            return c

        lax.fori_loop(0, CHUNK, zb, 0)
        base_r = sid * ROWS_PER_TILE
        for j in range(ROWS_PER_TILE // CHUNK):
            pltpu.sync_copy(msg[0], acc_sp.at[pl.ds(base_r + j * CHUNK, CHUNK)])
        pltpu.sync_copy(src_h.at[wid], sidx)
        pltpu.sync_copy(dst_h.at[wid], didx)
        plsc.subcore_barrier()

        def fire_gather(ci, p):
            pltpu.async_copy(t2_h.at[sidx.at[ci]], s2[p], semg[p])
            pltpu.async_copy(t2_h.at[didx.at[ci]], d2[p], semg[p])

        def wait_gather(ci, p):
            pltpu.make_async_copy(t2_h.at[sidx.at[ci]], s2[p], semg[p]).wait()
            pltpu.make_async_copy(t2_h.at[didx.at[ci]], d2[p], semg[p]).wait()

        def fire_scatter(ci, p):
            pltpu.async_copy(msg[p], acc_sp.at[didx.at[ci]], sems[p], add=True)

        def wait_scatter(ci, p):
            pltpu.make_async_copy(msg[p], acc_sp.at[didx.at[ci]], sems[p]).wait()

        def compute(p):
            @plsc.parallel_loop(0, CHUNK // 16, unroll=4)
            def wbody(t):
                rvec = t * 16 + iota
                a_s = plsc.load_gather(s2[p], [rvec, col8])
                a_d = plsc.load_gather(d2[p], [rvec, col9])
                e = a_s + a_d
                e = jnp.where(e >= 0.0, e, 0.2 * e)
                wv[p][pl.ds(t * 16, 16)] = jnp.exp(e)

            @plsc.parallel_loop(0, CHUNK, unroll=8)
            def mbody(b):
                bfull = jnp.full((16,), 0, jnp.int32) + b
                srow = s2[p][b, pl.ds(0, 16)]
                w16 = plsc.load_gather(wv[p], [bfull])
                msg[p][b, pl.ds(0, 16)] = srow * w16

        for p in range(2):
            fire_gather(p, p)
        for p in range(2):
            wait_gather(p, p)
            compute(p)
            fire_scatter(p, p)
            fire_gather(p + 2, p)

        def outer(g, c):
            for p in range(2):
                ci = 2 * g + p
                wait_gather(ci, p)
                wait_scatter(ci - 2, p)
                compute(p)
                fire_scatter(ci, p)
                fire_gather(ci + 2, p)
            return c

        lax.fori_loop(1, N_CHUNKS // 2 - 1, outer, 0)
        for p in range(2):
            ci = N_CHUNKS - 2 + p
            wait_gather(ci, p)
            wait_scatter(ci - 2, p)
            compute(p)
            fire_scatter(ci, p)
        for p in range(2):
            wait_scatter(N_CHUNKS - 2 + p, p)
        plsc.subcore_barrier()
        pltpu.sync_copy(acc_sp.at[pl.ds(base_r, ROWS_PER_TILE)],
                        acc_h.at[cid, pl.ds(base_r, ROWS_PER_TILE)])

    return k(src, dst, t2_tab)


def _stage_e(acc2, bias2p):
    def body(acc_ref, b2_ref, o_ref):
        o = acc_ref[0] + acc_ref[1]
        d = o[:, 7:8]
        v = o / (d + 1e-16) + b2_ref[...]
        lane = lax.broadcasted_iota(jnp.int32, (BLK, 16), 1)
        v = jnp.where(lane < C_OUT, v, -1e30)
        m = jnp.max(v, axis=1, keepdims=True)
        s = jnp.log(jnp.sum(jnp.exp(v - m), axis=1, keepdims=True))
        o_ref[...] = v - m - s

    return pl.pallas_call(
        body,
        grid=(N_PAD // BLK,),
        in_specs=[
            pl.BlockSpec((2, BLK, 16), lambda i: (0, i, 0)),
            pl.BlockSpec((1, 16), lambda i: (0, 0)),
        ],
        out_specs=pl.BlockSpec((BLK, 16), lambda i: (i, 0)),
        out_shape=jax.ShapeDtypeStruct((N_PAD, 16), jnp.float32),
    )(acc2, bias2p)


def kernel(x, edge_index, W1, att_src1, att_dst1, bias1,
           W2, att_src2, att_dst2, bias2):
    loop = jnp.arange(N_NODES, dtype=jnp.int32)
    npad = E_PAD - E_REAL
    pad_idx = N_NODES + (jnp.arange(npad, dtype=jnp.int32) % (N_PAD - N_NODES))
    src = jnp.concatenate([edge_index[0].astype(jnp.int32), loop, pad_idx]
                          ).reshape(N_CHUNKS, NW, CHUNK).swapaxes(0, 1)
    dst = jnp.concatenate([edge_index[1].astype(jnp.int32), loop, pad_idx]
                          ).reshape(N_CHUNKS, NW, CHUNK).swapaxes(0, 1)
    x_pad = jnp.zeros((N_PAD, D_IN), jnp.float32).at[:N_NODES].set(x)

    w1r = W1.reshape(D_IN, H1, F1)
    a_src_w = jnp.einsum("dhf,hf->dh", w1r, att_src1)
    a_dst_w = jnp.einsum("dhf,hf->dh", w1r, att_dst1)
    zpad8 = jnp.zeros((D_IN, 8), jnp.float32)
    w1s = jnp.concatenate([W1, a_src_w, zpad8], axis=1)
    w1d = jnp.concatenate([a_dst_w, zpad8], axis=1)

    s_tab, d_tab = _stage_a(x_pad, w1s, w1d)
    acc = _sc_layer1(src, dst, s_tab, d_tab)

    r16 = jnp.zeros((16, 64), jnp.float32).at[:8].set(
        jnp.repeat(jnp.eye(8, dtype=jnp.float32), 8, axis=1))
    w2c = jnp.zeros((64, 16), jnp.float32)
    w2c = w2c.at[:, :C_OUT].set(W2)
    w2c = w2c.at[:, 8].set(W2 @ att_src2[0])
    w2c = w2c.at[:, 9].set(W2 @ att_dst2[0])
    e7 = jnp.zeros((1, 16), jnp.float32).at[0, 7].set(1.0)

    t2_tab = _stage_c(acc, bias1.reshape(1, 64), r16, w2c, e7)
    acc2 = _sc_layer2(src, dst, t2_tab)
    out = _stage_e(acc2, jnp.pad(bias2, (0, 16 - C_OUT)).reshape(1, 16))
    return out[:N_NODES, :C_OUT]
